# baseline (device time: 250098 ns/iter reference)
import jax
import jax.numpy as jnp
from jax import lax
from jax.experimental import pallas as pl
from jax.experimental.pallas import tpu as pltpu

NC = 512


def _fused(o_mine, o_other, Wo):
    s_half, k = o_mine.shape
    n = Wo.shape[1]
    n_chunks = n // NC
    grid = (n_chunks + 1,)

    def body(o_mine_ref, o_other_ref, wo_ref, out_ref,
             send_buf, recv_buf, mine_buf, wo_bf,
             send_sems, recv_sems, credit_sem):
        i = pl.program_id(0)
        my_x = lax.axis_index("x")
        my_y = lax.axis_index("y")
        my_z = lax.axis_index("z")
        partner = (my_x, 1 - my_y, my_z)
        slot = lax.rem(i, 2)
        prev_slot = lax.rem(i + 1, 2)

        def desc(s):
            return pltpu.make_async_remote_copy(
                src_ref=send_buf.at[s],
                dst_ref=recv_buf.at[s],
                send_sem=send_sems.at[s],
                recv_sem=recv_sems.at[s],
                device_id=partner,
                device_id_type=pl.DeviceIdType.MESH,
            )

        @pl.when(i == 0)
        def _():
            barrier_sem = pltpu.get_barrier_semaphore()
            pl.semaphore_signal(
                barrier_sem, inc=1, device_id=partner,
                device_id_type=pl.DeviceIdType.MESH,
            )
            pl.semaphore_wait(barrier_sem, 1)

        @pl.when(i < n_chunks)
        def _compute_and_send():
            @pl.when(i >= 2)
            def _():
                desc(slot).wait_send()
                pl.semaphore_wait(credit_sem, 1)

            wo_bf[...] = wo_ref[...].astype(jnp.bfloat16)
            send_buf[slot] = jnp.dot(
                o_other_ref[...], wo_bf[...],
                preferred_element_type=jnp.float32,
            ).astype(jnp.bfloat16)
            desc(slot).start()
            mine_buf[slot] = jnp.dot(
                o_mine_ref[...], wo_bf[...],
                preferred_element_type=jnp.float32,
            )

        @pl.when(i >= 1)
        def _consume_prev():
            desc(prev_slot).wait_recv()
            out_ref[...] = (
                mine_buf[prev_slot]
                + recv_buf[prev_slot].astype(jnp.float32)
            )
            @pl.when(i <= n_chunks - 2)
            def _():
                pl.semaphore_signal(
                    credit_sem, inc=1, device_id=partner,
                    device_id_type=pl.DeviceIdType.MESH,
                )

        @pl.when(i == n_chunks)
        def _drain():
            for s in range(2):
                desc(s).wait_send()

    return pl.pallas_call(
        body,
        grid=grid,
        out_shape=jax.ShapeDtypeStruct((s_half, n), jnp.float32),
        in_specs=[
            pl.BlockSpec((s_half, k), lambda i: (0, 0),
                         memory_space=pltpu.VMEM),
            pl.BlockSpec((s_half, k), lambda i: (0, 0),
                         memory_space=pltpu.VMEM),
            pl.BlockSpec((k, NC), lambda i: (0, jnp.minimum(i, n // NC - 1)),
                         memory_space=pltpu.VMEM),
        ],
        out_specs=pl.BlockSpec((s_half, NC),
                               lambda i: (0, jnp.maximum(i - 1, 0)),
                               memory_space=pltpu.VMEM),
        scratch_shapes=[
            pltpu.VMEM((2, s_half, NC), jnp.bfloat16),
            pltpu.VMEM((2, s_half, NC), jnp.bfloat16),
            pltpu.VMEM((2, s_half, NC), jnp.float32),
            pltpu.VMEM((k, NC), jnp.bfloat16),
            pltpu.SemaphoreType.DMA((2,)),
            pltpu.SemaphoreType.DMA((2,)),
            pltpu.SemaphoreType.REGULAR,
        ],
        compiler_params=pltpu.CompilerParams(
            collective_id=0,
            dimension_semantics=("arbitrary",),
            vmem_limit_bytes=64 * 1024 * 1024,
        ),
    )(o_mine, o_other, Wo)


def kernel(O, Wo):
    b, s, h, d = O.shape
    k = h * d
    n = Wo.shape[1]
    s_half = s // 2

    my_y = lax.axis_index("y")

    o2 = O.reshape(s, k).astype(jnp.bfloat16)
    o_mine = lax.dynamic_slice_in_dim(o2, my_y * s_half, s_half, axis=0)
    o_other = lax.dynamic_slice_in_dim(o2, (1 - my_y) * s_half, s_half, axis=0)

    out = _fused(o_mine, o_other, Wo)
    return out.reshape(b, s_half, n)


# device time: 245167 ns/iter; 1.0201x vs baseline; 1.0201x over previous
import jax
import jax.numpy as jnp
from jax import lax
from jax.experimental import pallas as pl
from jax.experimental.pallas import tpu as pltpu

NC = 512


def _fused(o2, Wo):
    s, k = o2.shape
    s_half = s // 2
    n = Wo.shape[1]
    n_chunks = n // NC
    grid = (n_chunks + 1,)

    def body(o_ref, wo_ref, out_ref,
             send_buf, recv_buf, mine_buf, wo_bf,
             send_sems, recv_sems, credit_sem):
        i = pl.program_id(0)
        my_x = lax.axis_index("x")
        my_y = lax.axis_index("y")
        my_z = lax.axis_index("z")
        partner = (my_x, 1 - my_y, my_z)
        slot = lax.rem(i, 2)
        prev_slot = lax.rem(i + 1, 2)

        def desc(s):
            return pltpu.make_async_remote_copy(
                src_ref=send_buf.at[s],
                dst_ref=recv_buf.at[s],
                send_sem=send_sems.at[s],
                recv_sem=recv_sems.at[s],
                device_id=partner,
                device_id_type=pl.DeviceIdType.MESH,
            )

        @pl.when(i == 0)
        def _():
            barrier_sem = pltpu.get_barrier_semaphore()
            pl.semaphore_signal(
                barrier_sem, inc=1, device_id=partner,
                device_id_type=pl.DeviceIdType.MESH,
            )
            pl.semaphore_wait(barrier_sem, 1)

        @pl.when(i < n_chunks)
        def _compute_and_send():
            @pl.when(i >= 2)
            def _():
                desc(slot).wait_send()
                pl.semaphore_wait(credit_sem, 1)

            wo_bf[...] = wo_ref[...].astype(jnp.bfloat16)
            o_other = o_ref[pl.ds((1 - my_y) * s_half, s_half), :]
            send_buf[slot] = jnp.dot(
                o_other, wo_bf[...],
                preferred_element_type=jnp.float32,
            ).astype(jnp.bfloat16)
            desc(slot).start()
            o_mine = o_ref[pl.ds(my_y * s_half, s_half), :]
            mine_buf[slot] = jnp.dot(
                o_mine, wo_bf[...],
                preferred_element_type=jnp.float32,
            )

        @pl.when(i >= 1)
        def _consume_prev():
            desc(prev_slot).wait_recv()
            out_ref[...] = (
                mine_buf[prev_slot]
                + recv_buf[prev_slot].astype(jnp.float32)
            )
            @pl.when(i <= n_chunks - 2)
            def _():
                pl.semaphore_signal(
                    credit_sem, inc=1, device_id=partner,
                    device_id_type=pl.DeviceIdType.MESH,
                )

        @pl.when(i == n_chunks)
        def _drain():
            for s in range(2):
                desc(s).wait_send()

    return pl.pallas_call(
        body,
        grid=grid,
        out_shape=jax.ShapeDtypeStruct((s_half, n), jnp.float32),
        in_specs=[
            pl.BlockSpec((s, k), lambda i: (0, 0),
                         memory_space=pltpu.VMEM),
            pl.BlockSpec((k, NC), lambda i: (0, jnp.minimum(i, n // NC - 1)),
                         memory_space=pltpu.VMEM),
        ],
        out_specs=pl.BlockSpec((s_half, NC),
                               lambda i: (0, jnp.maximum(i - 1, 0)),
                               memory_space=pltpu.VMEM),
        scratch_shapes=[
            pltpu.VMEM((2, s_half, NC), jnp.bfloat16),
            pltpu.VMEM((2, s_half, NC), jnp.bfloat16),
            pltpu.VMEM((2, s_half, NC), jnp.float32),
            pltpu.VMEM((k, NC), jnp.bfloat16),
            pltpu.SemaphoreType.DMA((2,)),
            pltpu.SemaphoreType.DMA((2,)),
            pltpu.SemaphoreType.REGULAR,
        ],
        compiler_params=pltpu.CompilerParams(
            collective_id=0,
            dimension_semantics=("arbitrary",),
            vmem_limit_bytes=64 * 1024 * 1024,
        ),
    )(o2, Wo)


def kernel(O, Wo):
    b, s, h, d = O.shape
    k = h * d
    n = Wo.shape[1]
    s_half = s // 2

    o2 = O.reshape(s, k).astype(jnp.bfloat16)
    out = _fused(o2, Wo)
    return out.reshape(b, s_half, n)


# device time: 244805 ns/iter; 1.0216x vs baseline; 1.0015x over previous
import jax
import jax.numpy as jnp
from jax import lax
from jax.experimental import pallas as pl
from jax.experimental.pallas import tpu as pltpu

NC = 512


def _fused(o2, Wo):
    s, k = o2.shape
    s_half = s // 2
    n = Wo.shape[1]
    n_chunks = n // NC
    grid = (n_chunks + 1,)

    def body(o_ref, wo_ref, out_ref,
             send_buf, recv_buf, mine_buf, wo_bf,
             send_sems, recv_sems, credit_sem):
        i = pl.program_id(0)
        my_x = lax.axis_index("x")
        my_y = lax.axis_index("y")
        my_z = lax.axis_index("z")
        partner = (my_x, 1 - my_y, my_z)
        slot = lax.rem(i, 2)
        prev_slot = lax.rem(i + 1, 2)

        def desc(s):
            return pltpu.make_async_remote_copy(
                src_ref=send_buf.at[s],
                dst_ref=recv_buf.at[s],
                send_sem=send_sems.at[s],
                recv_sem=recv_sems.at[s],
                device_id=partner,
                device_id_type=pl.DeviceIdType.MESH,
            )

        @pl.when(i == 0)
        def _():
            barrier_sem = pltpu.get_barrier_semaphore()
            pl.semaphore_signal(
                barrier_sem, inc=1, device_id=partner,
                device_id_type=pl.DeviceIdType.MESH,
            )
            pl.semaphore_wait(barrier_sem, 1)

        @pl.when(i < n_chunks)
        def _compute_and_send():
            @pl.when(i >= 2)
            def _():
                desc(slot).wait_send()
                pl.semaphore_wait(credit_sem, 1)

            wo_bf[...] = wo_ref[...].astype(jnp.bfloat16)
            o_other = o_ref[pl.ds((1 - my_y) * s_half, s_half), :]
            send_buf[slot] = jnp.dot(
                o_other, wo_bf[...],
                preferred_element_type=jnp.float32,
            ).astype(jnp.bfloat16)
            desc(slot).start()
            o_mine = o_ref[pl.ds(my_y * s_half, s_half), :]
            mine_buf[slot] = jnp.dot(
                o_mine, wo_bf[...],
                preferred_element_type=jnp.float32,
            )

        @pl.when(i >= 1)
        def _consume_prev():
            desc(prev_slot).wait_recv()
            out_ref[0, :, :] = (
                mine_buf[prev_slot]
                + recv_buf[prev_slot].astype(jnp.float32)
            )
            @pl.when(i <= n_chunks - 2)
            def _():
                pl.semaphore_signal(
                    credit_sem, inc=1, device_id=partner,
                    device_id_type=pl.DeviceIdType.MESH,
                )

        @pl.when(i == n_chunks)
        def _drain():
            for s in range(2):
                desc(s).wait_send()

    return pl.pallas_call(
        body,
        grid=grid,
        out_shape=jax.ShapeDtypeStruct((1, s_half, n), jnp.float32),
        in_specs=[
            pl.BlockSpec((s, k), lambda i: (0, 0),
                         memory_space=pltpu.VMEM),
            pl.BlockSpec((k, NC), lambda i: (0, jnp.minimum(i, n // NC - 1)),
                         memory_space=pltpu.VMEM),
        ],
        out_specs=pl.BlockSpec((1, s_half, NC),
                               lambda i: (0, 0, jnp.maximum(i - 1, 0)),
                               memory_space=pltpu.VMEM),
        scratch_shapes=[
            pltpu.VMEM((2, s_half, NC), jnp.bfloat16),
            pltpu.VMEM((2, s_half, NC), jnp.bfloat16),
            pltpu.VMEM((2, s_half, NC), jnp.float32),
            pltpu.VMEM((k, NC), jnp.bfloat16),
            pltpu.SemaphoreType.DMA((2,)),
            pltpu.SemaphoreType.DMA((2,)),
            pltpu.SemaphoreType.REGULAR,
        ],
        compiler_params=pltpu.CompilerParams(
            collective_id=0,
            dimension_semantics=("arbitrary",),
            vmem_limit_bytes=64 * 1024 * 1024,
        ),
    )(o2, Wo)


def kernel(O, Wo):
    b, s, h, d = O.shape
    k = h * d
    n = Wo.shape[1]
    s_half = s // 2

    o2 = O.reshape(s, k).astype(jnp.bfloat16)
    return _fused(o2, Wo)
